# Initial kernel scaffold; baseline (speedup 1.0000x reference)
import jax
import jax.numpy as jnp
from jax.experimental import pallas as pl


def kernel(inputs, edge_index, W1, b1, W2, b2):
    raise NotImplementedError("write your pallas kernel here")


# speedup vs baseline: 4.4816x; 4.4816x over previous
import jax
import jax.numpy as jnp
from jax import lax
from jax.experimental import pallas as pl
from jax.experimental.pallas import tpu as pltpu
from jax.experimental.pallas import tpu_sc as plsc

NN = 10000
NP = 10240
EE = 320000
NC = 2
NS = 16
NW = NC * NS
K = 128
NCHUNK = EE // K
RPT = NP // NS
DEGW = 8


def _tile_ids():
    cid = lax.axis_index("c")
    sid = lax.axis_index("s")
    wid = cid * NS + sid
    per, rem = NCHUNK // NW, NCHUNK % NW
    start = wid * per + jnp.minimum(wid, rem)
    cnt = per + (wid < rem).astype(jnp.int32)
    return cid, sid, start, cnt


def _deg_body(src_hbm, dst_hbm, out_hbm, sbuf, dbuf, tbl0, tbl1):
    cid, sid, start, cnt = _tile_ids()
    wid = cid * NS + sid
    zero16 = jnp.zeros((16,), jnp.float32)
    one16 = jnp.full((16,), 1.0, jnp.float32)

    def zstep(i, carry):
        tbl0[pl.ds(i * 16, 16)] = zero16
        tbl1[pl.ds(i * 16, 16)] = zero16
        return carry

    lax.fori_loop(0, NP // 16, zstep, 0)

    def step(i, carry):
        base = (start + i) * K
        pltpu.sync_copy(src_hbm.at[pl.ds(base, K)], sbuf)
        pltpu.sync_copy(dst_hbm.at[pl.ds(base, K)], dbuf)
        for j in range(K // 16):
            sv = sbuf[pl.ds(j * 16, 16)]
            dv = dbuf[pl.ds(j * 16, 16)]
            plsc.addupdate_scatter(tbl0, [sv], one16)
            plsc.addupdate_scatter(tbl1, [dv], one16)
        return carry

    lax.fori_loop(0, cnt, step, 0)
    pltpu.sync_copy(tbl0, out_hbm.at[pl.ds(wid * NP, NP)])
    pltpu.sync_copy(tbl1, out_hbm.at[pl.ds((NW + wid) * NP, NP)])


def _sc_degrees(src, dst):
    k = pl.kernel(
        _deg_body,
        out_type=jax.ShapeDtypeStruct((2 * NW * NP,), jnp.float32),
        mesh=plsc.VectorSubcoreMesh(core_axis_name="c", subcore_axis_name="s"),
        compiler_params=pltpu.CompilerParams(needs_layout_passes=False),
        scratch_types=[
            pltpu.VMEM((K,), jnp.int32),
            pltpu.VMEM((K,), jnp.int32),
            pltpu.VMEM((NP,), jnp.float32),
            pltpu.VMEM((NP,), jnp.float32),
        ],
    )
    return k(src, dst).reshape(2, NW, NP)


def _agg_body(h_hbm, src_hbm, dst_hbm, zeros_hbm, out_hbm, sidx, didx, rows, agg_sh):
    cid, sid, start, cnt = _tile_ids()
    r0 = sid * RPT
    pltpu.sync_copy(zeros_hbm.at[pl.ds(r0, RPT)], agg_sh.at[pl.ds(r0, RPT)])
    plsc.subcore_barrier()

    def step(i, carry):
        base = (start + i) * K
        pltpu.sync_copy(src_hbm.at[pl.ds(base, K)], sidx.at[0])
        pltpu.sync_copy(dst_hbm.at[pl.ds(base, K)], didx.at[0])
        pltpu.sync_copy(h_hbm.at[sidx.at[0]], rows)
        pltpu.sync_copy(rows, agg_sh.at[didx.at[0]], add=True)
        return carry

    lax.fori_loop(0, cnt, step, 0)
    plsc.subcore_barrier()
    pltpu.sync_copy(agg_sh.at[pl.ds(r0, RPT)], out_hbm.at[cid].at[pl.ds(r0, RPT)])


def _sc_aggregate(h, src, dst):
    zeros = jnp.zeros((NP, 128), jnp.float32)
    k = pl.kernel(
        _agg_body,
        out_type=jax.ShapeDtypeStruct((NC, NP, 128), jnp.float32),
        mesh=plsc.VectorSubcoreMesh(core_axis_name="c", subcore_axis_name="s"),
        scratch_types=[
            pltpu.VMEM((1, K), jnp.int32),
            pltpu.VMEM((1, K), jnp.int32),
            pltpu.VMEM((K, 128), jnp.float32),
            pltpu.VMEM_SHARED((NP, 128), jnp.float32),
        ],
    )
    return k(h, src, dst, zeros)


def _nsrc(deg):
    d = jnp.sum(deg[0, :NN, :], axis=-1, keepdims=True)
    return lax.rsqrt(jnp.maximum(d, 1.0))


def _ndst(deg):
    d = jnp.sum(deg[1, :NN, :], axis=-1, keepdims=True)
    return lax.rsqrt(jnp.maximum(d, 1.0))


def _t1_body(deg_ref, x_ref, w1_ref, out_ref):
    deg = deg_ref[...]
    out_ref[...] = jnp.dot(x_ref[...] * _nsrc(deg), w1_ref[...],
                           preferred_element_type=jnp.float32)


def _t2_body(deg_ref, p_ref, b1_ref, out_ref):
    deg = deg_ref[...]
    agg = p_ref[0, :NN, :] + p_ref[1, :NN, :]
    h1 = jnp.maximum(agg * _ndst(deg) + b1_ref[...][None, :], 0.0)
    out_ref[...] = h1 * _nsrc(deg)


def _t3_body(deg_ref, p_ref, b2_ref, w2_ref, out_ref):
    deg = deg_ref[...]
    agg = p_ref[0, :NN, :] + p_ref[1, :NN, :]
    pre = jnp.dot(agg, w2_ref[...], preferred_element_type=jnp.float32)
    out_ref[...] = jnp.maximum(pre * _ndst(deg) + b2_ref[...][None, :], 0.0)


def _tc_call(body, out_shape, *args):
    return pl.pallas_call(body, out_shape=out_shape)(*args)


BM = 400


def _adj_body(a_ref, b_ref, out_ref):
    out_ref[...] = lax.dot_general(
        a_ref[...], b_ref[...], (((1,), (1,)), ((), ())),
        preferred_element_type=jnp.float32)


def _adj(h2):
    return pl.pallas_call(
        _adj_body,
        grid=(NN // BM,),
        in_specs=[pl.BlockSpec((BM, 64), lambda i: (i, 0)),
                  pl.BlockSpec((NN, 64), lambda i: (0, 0))],
        out_specs=pl.BlockSpec((BM, NN), lambda i: (i, 0)),
        out_shape=jax.ShapeDtypeStruct((NN, NN), jnp.float32),
    )(h2, h2)


def kernel(inputs, edge_index, W1, b1, W2, b2):
    src = edge_index[0]
    dst = edge_index[1]
    deg = jnp.transpose(_sc_degrees(src, dst), (0, 2, 1))
    h1pre = _tc_call(_t1_body, jax.ShapeDtypeStruct((NN, 128), jnp.float32),
                     deg, inputs, W1)
    p1 = _sc_aggregate(h1pre, src, dst)
    h1n = _tc_call(_t2_body, jax.ShapeDtypeStruct((NN, 128), jnp.float32),
                   deg, p1, b1)
    p2 = _sc_aggregate(h1n, src, dst)
    h2 = _tc_call(_t3_body, jax.ShapeDtypeStruct((NN, 64), jnp.float32),
                  deg, p2, b2, W2)
    adj = _adj(h2)
    return (adj, h2)


# speedup vs baseline: 6.9907x; 1.5599x over previous
import jax
import jax.numpy as jnp
from jax import lax
from jax.experimental import pallas as pl
from jax.experimental.pallas import tpu as pltpu
from jax.experimental.pallas import tpu_sc as plsc

NN = 10000
NP = 10240
EE = 320000
NC = 2
NS = 16
NW = NC * NS
K = 128
NCHUNK = EE // K
RPT = NP // NS
DEGW = 8


def _tile_ids():
    cid = lax.axis_index("c")
    sid = lax.axis_index("s")
    wid = cid * NS + sid
    per, rem = NCHUNK // NW, NCHUNK % NW
    start = wid * per + jnp.minimum(wid, rem)
    cnt = per + (wid < rem).astype(jnp.int32)
    return cid, sid, start, cnt


def _deg_body(src_hbm, dst_hbm, out_hbm, sbuf, dbuf, tbl0, tbl1):
    cid, sid, start, cnt = _tile_ids()
    wid = cid * NS + sid
    zero16 = jnp.zeros((16,), jnp.float32)
    one16 = jnp.full((16,), 1.0, jnp.float32)

    def zstep(i, carry):
        tbl0[pl.ds(i * 16, 16)] = zero16
        tbl1[pl.ds(i * 16, 16)] = zero16
        return carry

    lax.fori_loop(0, NP // 16, zstep, 0)

    def step(i, carry):
        base = (start + i) * K
        pltpu.sync_copy(src_hbm.at[pl.ds(base, K)], sbuf)
        pltpu.sync_copy(dst_hbm.at[pl.ds(base, K)], dbuf)
        for j in range(K // 16):
            sv = sbuf[pl.ds(j * 16, 16)]
            dv = dbuf[pl.ds(j * 16, 16)]
            plsc.addupdate_scatter(tbl0, [sv], one16)
            plsc.addupdate_scatter(tbl1, [dv], one16)
        return carry

    lax.fori_loop(0, cnt, step, 0)
    pltpu.sync_copy(tbl0, out_hbm.at[pl.ds(wid * NP, NP)])
    pltpu.sync_copy(tbl1, out_hbm.at[pl.ds((NW + wid) * NP, NP)])


def _sc_degrees(src, dst):
    k = pl.kernel(
        _deg_body,
        out_type=jax.ShapeDtypeStruct((2 * NW * NP,), jnp.float32),
        mesh=plsc.VectorSubcoreMesh(core_axis_name="c", subcore_axis_name="s"),
        compiler_params=pltpu.CompilerParams(needs_layout_passes=False),
        scratch_types=[
            pltpu.VMEM((K,), jnp.int32),
            pltpu.VMEM((K,), jnp.int32),
            pltpu.VMEM((NP,), jnp.float32),
            pltpu.VMEM((NP,), jnp.float32),
        ],
    )
    return k(src, dst).reshape(2, NW, NP)


def _agg_body(h_hbm, src_hbm, dst_hbm, zeros_hbm, out_hbm,
              sidx, didx, rows,
              gs0, gs1, ss0, ss1, is0, is1, is2, is3, agg_sh):
    gsem = (gs0, gs1)
    ssem = (ss0, ss1)
    isem = (is0, is1, is2, is3)
    cid, sid, start, cnt = _tile_ids()
    r0 = sid * RPT

    def idx_start(t, i):
        pltpu.async_copy(src_hbm.at[pl.ds((start + t) * K, K)], sidx.at[i], isem[i])
        pltpu.async_copy(dst_hbm.at[pl.ds((start + t) * K, K)], didx.at[i], isem[i])

    def idx_wait(t, i):
        pltpu.make_async_copy(src_hbm.at[pl.ds((start + t) * K, K)], sidx.at[i],
                              isem[i]).wait()
        pltpu.make_async_copy(dst_hbm.at[pl.ds((start + t) * K, K)], didx.at[i],
                              isem[i]).wait()

    pltpu.sync_copy(zeros_hbm.at[pl.ds(r0, RPT)], agg_sh.at[pl.ds(r0, RPT)])
    idx_start(0, 0)
    idx_start(1, 1)
    plsc.subcore_barrier()

    def superstep(s, carry):
        for b in range(4):
            t = s * 4 + b
            r = b % 2
            pr = (b - 1) % 2
            pi = (b - 1) % 4
            ni = (b + 2) % 4

            @pl.when(t < cnt)
            def _launch():
                @pl.when(t >= 2)
                def _free():
                    pltpu.make_async_copy(rows.at[r], agg_sh.at[didx.at[0]],
                                          ssem[r]).wait()
                idx_wait(t, b)
                pltpu.async_copy(h_hbm.at[sidx.at[b]], rows.at[r], gsem[r])

                @pl.when(t + 2 < cnt)
                def _prefetch():
                    idx_start(t + 2, ni)

            @pl.when(jnp.logical_and(t >= 1, t <= cnt))
            def _consume():
                pltpu.make_async_copy(h_hbm.at[sidx.at[pi]], rows.at[pr],
                                      gsem[pr]).wait()
                pltpu.async_copy(rows.at[pr], agg_sh.at[didx.at[pi]],
                                 ssem[pr], add=True)
        return carry

    lax.fori_loop(0, (cnt + 4) // 4, superstep, 0)
    for r in range(2):
        pltpu.make_async_copy(rows.at[r], agg_sh.at[didx.at[0]], ssem[r]).wait()
    plsc.subcore_barrier()
    pltpu.sync_copy(agg_sh.at[pl.ds(r0, RPT)], out_hbm.at[cid].at[pl.ds(r0, RPT)])


def _sc_aggregate(h, src, dst):
    zeros = jnp.zeros((NP, 128), jnp.float32)
    k = pl.kernel(
        _agg_body,
        out_type=jax.ShapeDtypeStruct((NC, NP, 128), jnp.float32),
        mesh=plsc.VectorSubcoreMesh(core_axis_name="c", subcore_axis_name="s"),
        scratch_types=[
            pltpu.VMEM((4, K), jnp.int32),
            pltpu.VMEM((4, K), jnp.int32),
            pltpu.VMEM((2, K, 128), jnp.float32),
            pltpu.SemaphoreType.DMA,
            pltpu.SemaphoreType.DMA,
            pltpu.SemaphoreType.DMA,
            pltpu.SemaphoreType.DMA,
            pltpu.SemaphoreType.DMA,
            pltpu.SemaphoreType.DMA,
            pltpu.SemaphoreType.DMA,
            pltpu.SemaphoreType.DMA,
            pltpu.VMEM_SHARED((NP, 128), jnp.float32),
        ],
    )
    return k(h, src, dst, zeros)


def _nsrc(deg):
    d = jnp.sum(deg[0, :NN, :], axis=-1, keepdims=True)
    return lax.rsqrt(jnp.maximum(d, 1.0))


def _ndst(deg):
    d = jnp.sum(deg[1, :NN, :], axis=-1, keepdims=True)
    return lax.rsqrt(jnp.maximum(d, 1.0))


def _t1_body(deg_ref, x_ref, w1_ref, out_ref):
    deg = deg_ref[...]
    out_ref[...] = jnp.dot(x_ref[...] * _nsrc(deg), w1_ref[...],
                           preferred_element_type=jnp.float32)


def _t2_body(deg_ref, p_ref, b1_ref, out_ref):
    deg = deg_ref[...]
    agg = p_ref[0, :NN, :] + p_ref[1, :NN, :]
    h1 = jnp.maximum(agg * _ndst(deg) + b1_ref[...][None, :], 0.0)
    out_ref[...] = h1 * _nsrc(deg)


def _t3_body(deg_ref, p_ref, b2_ref, w2_ref, out_ref):
    deg = deg_ref[...]
    agg = p_ref[0, :NN, :] + p_ref[1, :NN, :]
    pre = jnp.dot(agg, w2_ref[...], preferred_element_type=jnp.float32)
    out_ref[...] = jnp.maximum(pre * _ndst(deg) + b2_ref[...][None, :], 0.0)


def _tc_call(body, out_shape, *args):
    return pl.pallas_call(body, out_shape=out_shape)(*args)


BM = 400


def _adj_body(a_ref, b_ref, out_ref):
    out_ref[...] = lax.dot_general(
        a_ref[...], b_ref[...], (((1,), (1,)), ((), ())),
        preferred_element_type=jnp.float32)


def _adj(h2):
    return pl.pallas_call(
        _adj_body,
        grid=(NN // BM,),
        in_specs=[pl.BlockSpec((BM, 64), lambda i: (i, 0)),
                  pl.BlockSpec((NN, 64), lambda i: (0, 0))],
        out_specs=pl.BlockSpec((BM, NN), lambda i: (i, 0)),
        out_shape=jax.ShapeDtypeStruct((NN, NN), jnp.float32),
    )(h2, h2)


def kernel(inputs, edge_index, W1, b1, W2, b2):
    src = edge_index[0]
    dst = edge_index[1]
    deg = jnp.transpose(_sc_degrees(src, dst), (0, 2, 1))
    h1pre = _tc_call(_t1_body, jax.ShapeDtypeStruct((NN, 128), jnp.float32),
                     deg, inputs, W1)
    p1 = _sc_aggregate(h1pre, src, dst)
    h1n = _tc_call(_t2_body, jax.ShapeDtypeStruct((NN, 128), jnp.float32),
                   deg, p1, b1)
    p2 = _sc_aggregate(h1n, src, dst)
    h2 = _tc_call(_t3_body, jax.ShapeDtypeStruct((NN, 64), jnp.float32),
                  deg, p2, b2, W2)
    adj = _adj(h2)
    return (adj, h2)


# speedup vs baseline: 7.9064x; 1.1310x over previous
import jax
import jax.numpy as jnp
from jax import lax
from jax.experimental import pallas as pl
from jax.experimental.pallas import tpu as pltpu
from jax.experimental.pallas import tpu_sc as plsc

NN = 10000
NP = 10240
EE = 320000
NC = 2
NS = 16
NW = NC * NS
K = 128
NCHUNK = EE // K
RPT = NP // NS
DEGW = 8


def _tile_ids():
    cid = lax.axis_index("c")
    sid = lax.axis_index("s")
    wid = cid * NS + sid
    per, rem = NCHUNK // NW, NCHUNK % NW
    start = wid * per + jnp.minimum(wid, rem)
    cnt = per + (wid < rem).astype(jnp.int32)
    return cid, sid, start, cnt


def _deg_body(src_hbm, dst_hbm, out_hbm, sbuf, dbuf, tbl0, tbl1,
              is0, is1, is2, is3):
    isem = (is0, is1, is2, is3)
    cid, sid, start, cnt = _tile_ids()
    wid = cid * NS + sid
    zero16 = jnp.zeros((16,), jnp.float32)
    one16 = jnp.full((16,), 1.0, jnp.float32)

    def idx_start(t, i):
        pltpu.async_copy(src_hbm.at[pl.ds((start + t) * K, K)], sbuf.at[i], isem[i])
        pltpu.async_copy(dst_hbm.at[pl.ds((start + t) * K, K)], dbuf.at[i], isem[i])

    def idx_wait(t, i):
        pltpu.make_async_copy(src_hbm.at[pl.ds((start + t) * K, K)], sbuf.at[i],
                              isem[i]).wait()
        pltpu.make_async_copy(dst_hbm.at[pl.ds((start + t) * K, K)], dbuf.at[i],
                              isem[i]).wait()

    idx_start(0, 0)
    idx_start(1, 1)

    def zstep(i, carry):
        tbl0[pl.ds(i * 16, 16)] = zero16
        tbl1[pl.ds(i * 16, 16)] = zero16
        return carry

    lax.fori_loop(0, NP // 16, zstep, 0)

    def superstep(s, carry):
        for b in range(4):
            t = s * 4 + b
            ni = (b + 2) % 4

            @pl.when(t < cnt)
            def _proc():
                idx_wait(t, b)

                @pl.when(t + 2 < cnt)
                def _prefetch():
                    idx_start(t + 2, ni)

                for j in range(K // 16):
                    sv = sbuf[b, pl.ds(j * 16, 16)]
                    dv = dbuf[b, pl.ds(j * 16, 16)]
                    plsc.addupdate_scatter(tbl0, [sv], one16)
                    plsc.addupdate_scatter(tbl1, [dv], one16)
        return carry

    lax.fori_loop(0, (cnt + 4) // 4, superstep, 0)
    pltpu.sync_copy(tbl0, out_hbm.at[pl.ds(wid * NP, NP)])
    pltpu.sync_copy(tbl1, out_hbm.at[pl.ds((NW + wid) * NP, NP)])


def _sc_degrees(src, dst):
    k = pl.kernel(
        _deg_body,
        out_type=jax.ShapeDtypeStruct((2 * NW * NP,), jnp.float32),
        mesh=plsc.VectorSubcoreMesh(core_axis_name="c", subcore_axis_name="s"),
        compiler_params=pltpu.CompilerParams(needs_layout_passes=False),
        scratch_types=[
            pltpu.VMEM((4, K), jnp.int32),
            pltpu.VMEM((4, K), jnp.int32),
            pltpu.VMEM((NP,), jnp.float32),
            pltpu.VMEM((NP,), jnp.float32),
            pltpu.SemaphoreType.DMA,
            pltpu.SemaphoreType.DMA,
            pltpu.SemaphoreType.DMA,
            pltpu.SemaphoreType.DMA,
        ],
    )
    return k(src, dst).reshape(2, NW, NP)


def _agg_body(h_hbm, src_hbm, dst_hbm, zeros_hbm, out_hbm,
              sidx, didx, rows,
              gs0, gs1, ss0, ss1, is0, is1, is2, is3, agg_sh):
    gsem = (gs0, gs1)
    ssem = (ss0, ss1)
    isem = (is0, is1, is2, is3)
    cid, sid, start, cnt = _tile_ids()
    r0 = sid * RPT

    def idx_start(t, i):
        pltpu.async_copy(src_hbm.at[pl.ds((start + t) * K, K)], sidx.at[i], isem[i])
        pltpu.async_copy(dst_hbm.at[pl.ds((start + t) * K, K)], didx.at[i], isem[i])

    def idx_wait(t, i):
        pltpu.make_async_copy(src_hbm.at[pl.ds((start + t) * K, K)], sidx.at[i],
                              isem[i]).wait()
        pltpu.make_async_copy(dst_hbm.at[pl.ds((start + t) * K, K)], didx.at[i],
                              isem[i]).wait()

    pltpu.sync_copy(zeros_hbm.at[pl.ds(r0, RPT)], agg_sh.at[pl.ds(r0, RPT)])
    idx_start(0, 0)
    idx_start(1, 1)
    plsc.subcore_barrier()

    def superstep(s, carry):
        for b in range(4):
            t = s * 4 + b
            r = b % 2
            pr = (b - 1) % 2
            pi = (b - 1) % 4
            ni = (b + 2) % 4

            @pl.when(t < cnt)
            def _launch():
                @pl.when(t >= 2)
                def _free():
                    pltpu.make_async_copy(rows.at[r], agg_sh.at[didx.at[0]],
                                          ssem[r]).wait()
                idx_wait(t, b)
                pltpu.async_copy(h_hbm.at[sidx.at[b]], rows.at[r], gsem[r])

                @pl.when(t + 2 < cnt)
                def _prefetch():
                    idx_start(t + 2, ni)

            @pl.when(jnp.logical_and(t >= 1, t <= cnt))
            def _consume():
                pltpu.make_async_copy(h_hbm.at[sidx.at[pi]], rows.at[pr],
                                      gsem[pr]).wait()
                pltpu.async_copy(rows.at[pr], agg_sh.at[didx.at[pi]],
                                 ssem[pr], add=True)
        return carry

    lax.fori_loop(0, (cnt + 4) // 4, superstep, 0)
    for r in range(2):
        pltpu.make_async_copy(rows.at[r], agg_sh.at[didx.at[0]], ssem[r]).wait()
    plsc.subcore_barrier()
    pltpu.sync_copy(agg_sh.at[pl.ds(r0, RPT)], out_hbm.at[cid].at[pl.ds(r0, RPT)])


def _sc_aggregate(h, src, dst):
    zeros = jnp.zeros((NP, 128), jnp.float32)
    k = pl.kernel(
        _agg_body,
        out_type=jax.ShapeDtypeStruct((NC, NP, 128), jnp.float32),
        mesh=plsc.VectorSubcoreMesh(core_axis_name="c", subcore_axis_name="s"),
        scratch_types=[
            pltpu.VMEM((4, K), jnp.int32),
            pltpu.VMEM((4, K), jnp.int32),
            pltpu.VMEM((2, K, 128), jnp.float32),
            pltpu.SemaphoreType.DMA,
            pltpu.SemaphoreType.DMA,
            pltpu.SemaphoreType.DMA,
            pltpu.SemaphoreType.DMA,
            pltpu.SemaphoreType.DMA,
            pltpu.SemaphoreType.DMA,
            pltpu.SemaphoreType.DMA,
            pltpu.SemaphoreType.DMA,
            pltpu.VMEM_SHARED((NP, 128), jnp.float32),
        ],
    )
    return k(h, src, dst, zeros)


def _nsrc(deg):
    d = jnp.sum(deg[0, :NN, :], axis=-1, keepdims=True)
    return lax.rsqrt(jnp.maximum(d, 1.0))


def _ndst(deg):
    d = jnp.sum(deg[1, :NN, :], axis=-1, keepdims=True)
    return lax.rsqrt(jnp.maximum(d, 1.0))


def _t1_body(deg_ref, x_ref, w1_ref, out_ref):
    deg = deg_ref[...]
    out_ref[...] = jnp.dot(x_ref[...] * _nsrc(deg), w1_ref[...],
                           preferred_element_type=jnp.float32)


def _t2_body(deg_ref, p_ref, b1_ref, out_ref):
    deg = deg_ref[...]
    agg = p_ref[0, :NN, :] + p_ref[1, :NN, :]
    h1 = jnp.maximum(agg * _ndst(deg) + b1_ref[...][None, :], 0.0)
    out_ref[...] = h1 * _nsrc(deg)


def _t3_body(deg_ref, p_ref, b2_ref, w2_ref, out_ref):
    deg = deg_ref[...]
    agg = p_ref[0, :NN, :] + p_ref[1, :NN, :]
    pre = jnp.dot(agg, w2_ref[...], preferred_element_type=jnp.float32)
    out_ref[...] = jnp.maximum(pre * _ndst(deg) + b2_ref[...][None, :], 0.0)


def _tc_call(body, out_shape, *args):
    return pl.pallas_call(body, out_shape=out_shape)(*args)


BM = 400


def _adj_body(a_ref, b_ref, out_ref):
    out_ref[...] = lax.dot_general(
        a_ref[...], b_ref[...], (((1,), (1,)), ((), ())),
        preferred_element_type=jnp.float32)


def _adj(h2):
    return pl.pallas_call(
        _adj_body,
        grid=(NN // BM,),
        in_specs=[pl.BlockSpec((BM, 64), lambda i: (i, 0)),
                  pl.BlockSpec((NN, 64), lambda i: (0, 0))],
        out_specs=pl.BlockSpec((BM, NN), lambda i: (i, 0)),
        out_shape=jax.ShapeDtypeStruct((NN, NN), jnp.float32),
    )(h2, h2)


def kernel(inputs, edge_index, W1, b1, W2, b2):
    src = edge_index[0]
    dst = edge_index[1]
    deg = jnp.transpose(_sc_degrees(src, dst), (0, 2, 1))
    h1pre = _tc_call(_t1_body, jax.ShapeDtypeStruct((NN, 128), jnp.float32),
                     deg, inputs, W1)
    p1 = _sc_aggregate(h1pre, src, dst)
    h1n = _tc_call(_t2_body, jax.ShapeDtypeStruct((NN, 128), jnp.float32),
                   deg, p1, b1)
    p2 = _sc_aggregate(h1n, src, dst)
    h2 = _tc_call(_t3_body, jax.ShapeDtypeStruct((NN, 64), jnp.float32),
                  deg, p2, b2, W2)
    adj = _adj(h2)
    return (adj, h2)


# speedup vs baseline: 8.0298x; 1.0156x over previous
import jax
import jax.numpy as jnp
from jax import lax
from jax.experimental import pallas as pl
from jax.experimental.pallas import tpu as pltpu
from jax.experimental.pallas import tpu_sc as plsc

NN = 10000
NP = 10240
EE = 320000
NC = 2
NS = 16
NW = NC * NS
K = 128
NCHUNK = EE // K
RPT = NP // NS
DEGW = 8


def _tile_ids():
    cid = lax.axis_index("c")
    sid = lax.axis_index("s")
    wid = cid * NS + sid
    per, rem = NCHUNK // NW, NCHUNK % NW
    start = wid * per + jnp.minimum(wid, rem)
    cnt = per + (wid < rem).astype(jnp.int32)
    return cid, sid, start, cnt


def _deg_body(src_hbm, dst_hbm, out_hbm, sbuf, dbuf, tbl0, tbl1,
              is0, is1, is2, is3):
    isem = (is0, is1, is2, is3)
    cid, sid, start, cnt = _tile_ids()
    wid = cid * NS + sid
    zero16 = jnp.zeros((16,), jnp.float32)
    one16 = jnp.full((16,), 1.0, jnp.float32)

    def idx_start(t, i):
        pltpu.async_copy(src_hbm.at[pl.ds((start + t) * K, K)], sbuf.at[i], isem[i])
        pltpu.async_copy(dst_hbm.at[pl.ds((start + t) * K, K)], dbuf.at[i], isem[i])

    def idx_wait(t, i):
        pltpu.make_async_copy(src_hbm.at[pl.ds((start + t) * K, K)], sbuf.at[i],
                              isem[i]).wait()
        pltpu.make_async_copy(dst_hbm.at[pl.ds((start + t) * K, K)], dbuf.at[i],
                              isem[i]).wait()

    idx_start(0, 0)
    idx_start(1, 1)

    def zstep(i, carry):
        tbl0[pl.ds(i * 16, 16)] = zero16
        tbl1[pl.ds(i * 16, 16)] = zero16
        return carry

    lax.fori_loop(0, NP // 16, zstep, 0)

    def superstep(s, carry):
        for b in range(4):
            t = s * 4 + b
            ni = (b + 2) % 4

            @pl.when(t < cnt)
            def _proc():
                idx_wait(t, b)

                @pl.when(t + 2 < cnt)
                def _prefetch():
                    idx_start(t + 2, ni)

                for j in range(K // 16):
                    sv = sbuf[b, pl.ds(j * 16, 16)]
                    dv = dbuf[b, pl.ds(j * 16, 16)]
                    plsc.addupdate_scatter(tbl0, [sv], one16)
                    plsc.addupdate_scatter(tbl1, [dv], one16)
        return carry

    lax.fori_loop(0, (cnt + 4) // 4, superstep, 0)
    pltpu.sync_copy(tbl0, out_hbm.at[pl.ds(wid * NP, NP)])
    pltpu.sync_copy(tbl1, out_hbm.at[pl.ds((NW + wid) * NP, NP)])


def _sc_degrees(src, dst):
    k = pl.kernel(
        _deg_body,
        out_type=jax.ShapeDtypeStruct((2 * NW * NP,), jnp.float32),
        mesh=plsc.VectorSubcoreMesh(core_axis_name="c", subcore_axis_name="s"),
        compiler_params=pltpu.CompilerParams(needs_layout_passes=False),
        scratch_types=[
            pltpu.VMEM((4, K), jnp.int32),
            pltpu.VMEM((4, K), jnp.int32),
            pltpu.VMEM((NP,), jnp.float32),
            pltpu.VMEM((NP,), jnp.float32),
            pltpu.SemaphoreType.DMA,
            pltpu.SemaphoreType.DMA,
            pltpu.SemaphoreType.DMA,
            pltpu.SemaphoreType.DMA,
        ],
    )
    return k(src, dst).reshape(2, NW, NP)


def _agg_body(h_hbm, src_hbm, dst_hbm, zeros_hbm, out_hbm,
              sidx, didx, rows,
              gs0, gs1, ss0, ss1, is0, is1, is2, is3, agg_sh):
    gsem = (gs0, gs1)
    ssem = (ss0, ss1)
    isem = (is0, is1, is2, is3)
    cid, sid, start, cnt = _tile_ids()
    r0 = sid * RPT

    def idx_start(t, i):
        pltpu.async_copy(src_hbm.at[pl.ds((start + t) * K, K)], sidx.at[i], isem[i])
        pltpu.async_copy(dst_hbm.at[pl.ds((start + t) * K, K)], didx.at[i], isem[i])

    def idx_wait(t, i):
        pltpu.make_async_copy(src_hbm.at[pl.ds((start + t) * K, K)], sidx.at[i],
                              isem[i]).wait()
        pltpu.make_async_copy(dst_hbm.at[pl.ds((start + t) * K, K)], didx.at[i],
                              isem[i]).wait()

    pltpu.sync_copy(zeros_hbm.at[pl.ds(r0, RPT)], agg_sh.at[pl.ds(r0, RPT)])
    idx_start(0, 0)
    idx_start(1, 1)
    plsc.subcore_barrier()

    def superstep(s, carry):
        for b in range(4):
            t = s * 4 + b
            r = b % 2
            pr = (b - 1) % 2
            pi = (b - 1) % 4
            ni = (b + 2) % 4

            @pl.when(t < cnt)
            def _launch():
                @pl.when(t >= 2)
                def _free():
                    pltpu.make_async_copy(rows.at[r], agg_sh.at[didx.at[0]],
                                          ssem[r]).wait()
                idx_wait(t, b)
                pltpu.async_copy(h_hbm.at[sidx.at[b]], rows.at[r], gsem[r])

                @pl.when(t + 2 < cnt)
                def _prefetch():
                    idx_start(t + 2, ni)

            @pl.when(jnp.logical_and(t >= 1, t <= cnt))
            def _consume():
                pltpu.make_async_copy(h_hbm.at[sidx.at[pi]], rows.at[pr],
                                      gsem[pr]).wait()
                pltpu.async_copy(rows.at[pr], agg_sh.at[didx.at[pi]],
                                 ssem[pr], add=True)
        return carry

    lax.fori_loop(0, (cnt + 4) // 4, superstep, 0)
    for r in range(2):
        pltpu.make_async_copy(rows.at[r], agg_sh.at[didx.at[0]], ssem[r]).wait()
    plsc.subcore_barrier()
    pltpu.sync_copy(agg_sh.at[pl.ds(r0, RPT)], out_hbm.at[cid].at[pl.ds(r0, RPT)])


def _sc_aggregate(h, src, dst):
    zeros = jnp.zeros((NP, 128), jnp.float32)
    k = pl.kernel(
        _agg_body,
        out_type=jax.ShapeDtypeStruct((NC, NP, 128), jnp.float32),
        mesh=plsc.VectorSubcoreMesh(core_axis_name="c", subcore_axis_name="s"),
        scratch_types=[
            pltpu.VMEM((4, K), jnp.int32),
            pltpu.VMEM((4, K), jnp.int32),
            pltpu.VMEM((2, K, 128), jnp.float32),
            pltpu.SemaphoreType.DMA,
            pltpu.SemaphoreType.DMA,
            pltpu.SemaphoreType.DMA,
            pltpu.SemaphoreType.DMA,
            pltpu.SemaphoreType.DMA,
            pltpu.SemaphoreType.DMA,
            pltpu.SemaphoreType.DMA,
            pltpu.SemaphoreType.DMA,
            pltpu.VMEM_SHARED((NP, 128), jnp.float32),
        ],
    )
    return k(h, src, dst, zeros)


def _norm_col(deg_t):
    d = jnp.sum(deg_t, axis=0, keepdims=True)
    d = jnp.transpose(d)[:NN, :]
    return lax.rsqrt(jnp.maximum(d, 1.0))


def _nsrc(deg):
    return _norm_col(deg[0])


def _ndst(deg):
    return _norm_col(deg[1])


def _t1_body(deg_ref, x_ref, w1_ref, out_ref):
    deg = deg_ref[...]
    out_ref[...] = jnp.dot(x_ref[...] * _nsrc(deg), w1_ref[...],
                           preferred_element_type=jnp.float32)


def _t2_body(deg_ref, p_ref, b1_ref, out_ref):
    deg = deg_ref[...]
    agg = p_ref[0, :NN, :] + p_ref[1, :NN, :]
    h1 = jnp.maximum(agg * _ndst(deg) + b1_ref[...][None, :], 0.0)
    out_ref[...] = h1 * _nsrc(deg)


def _t3_body(deg_ref, p_ref, b2_ref, w2_ref, out_ref):
    deg = deg_ref[...]
    agg = p_ref[0, :NN, :] + p_ref[1, :NN, :]
    pre = jnp.dot(agg, w2_ref[...], preferred_element_type=jnp.float32)
    out_ref[...] = jnp.maximum(pre * _ndst(deg) + b2_ref[...][None, :], 0.0)


def _tc_call(body, out_shape, *args):
    return pl.pallas_call(body, out_shape=out_shape)(*args)


BM = 400


def _adj_body(a_ref, b_ref, out_ref):
    out_ref[...] = lax.dot_general(
        a_ref[...], b_ref[...], (((1,), (1,)), ((), ())),
        preferred_element_type=jnp.float32)


def _adj(h2):
    return pl.pallas_call(
        _adj_body,
        grid=(NN // BM,),
        in_specs=[pl.BlockSpec((BM, 64), lambda i: (i, 0)),
                  pl.BlockSpec((NN, 64), lambda i: (0, 0))],
        out_specs=pl.BlockSpec((BM, NN), lambda i: (i, 0)),
        out_shape=jax.ShapeDtypeStruct((NN, NN), jnp.float32),
    )(h2, h2)


def kernel(inputs, edge_index, W1, b1, W2, b2):
    src = edge_index[0]
    dst = edge_index[1]
    deg = _sc_degrees(src, dst)
    h1pre = _tc_call(_t1_body, jax.ShapeDtypeStruct((NN, 128), jnp.float32),
                     deg, inputs, W1)
    p1 = _sc_aggregate(h1pre, src, dst)
    h1n = _tc_call(_t2_body, jax.ShapeDtypeStruct((NN, 128), jnp.float32),
                   deg, p1, b1)
    p2 = _sc_aggregate(h1n, src, dst)
    h2 = _tc_call(_t3_body, jax.ShapeDtypeStruct((NN, 64), jnp.float32),
                  deg, p2, b2, W2)
    adj = _adj(h2)
    return (adj, h2)


# speedup vs baseline: 8.0332x; 1.0004x over previous
import jax
import jax.numpy as jnp
from jax import lax
from jax.experimental import pallas as pl
from jax.experimental.pallas import tpu as pltpu
from jax.experimental.pallas import tpu_sc as plsc

NN = 10000
NP = 10240
EE = 320000
NC = 2
NS = 16
NW = NC * NS
K = 128
NCHUNK = EE // K
RPT = NP // NS
DEGW = 8


def _tile_ids():
    cid = lax.axis_index("c")
    sid = lax.axis_index("s")
    wid = cid * NS + sid
    per, rem = NCHUNK // NW, NCHUNK % NW
    start = wid * per + jnp.minimum(wid, rem)
    cnt = per + (wid < rem).astype(jnp.int32)
    return cid, sid, start, cnt


def _deg_body(src_hbm, dst_hbm, out_hbm, sbuf, dbuf, tbl0, tbl1,
              is0, is1, is2, is3):
    isem = (is0, is1, is2, is3)
    cid, sid, start, cnt = _tile_ids()
    wid = cid * NS + sid
    zero16 = jnp.zeros((16,), jnp.float32)
    one16 = jnp.full((16,), 1.0, jnp.float32)

    def idx_start(t, i):
        pltpu.async_copy(src_hbm.at[pl.ds((start + t) * K, K)], sbuf.at[i], isem[i])
        pltpu.async_copy(dst_hbm.at[pl.ds((start + t) * K, K)], dbuf.at[i], isem[i])

    def idx_wait(t, i):
        pltpu.make_async_copy(src_hbm.at[pl.ds((start + t) * K, K)], sbuf.at[i],
                              isem[i]).wait()
        pltpu.make_async_copy(dst_hbm.at[pl.ds((start + t) * K, K)], dbuf.at[i],
                              isem[i]).wait()

    idx_start(0, 0)
    idx_start(1, 1)

    def zstep(i, carry):
        tbl0[pl.ds(i * 16, 16)] = zero16
        tbl1[pl.ds(i * 16, 16)] = zero16
        return carry

    lax.fori_loop(0, NP // 16, zstep, 0)

    def superstep(s, carry):
        for b in range(4):
            t = s * 4 + b
            ni = (b + 2) % 4

            @pl.when(t < cnt)
            def _proc():
                idx_wait(t, b)

                @pl.when(t + 2 < cnt)
                def _prefetch():
                    idx_start(t + 2, ni)

                for j in range(K // 16):
                    sv = sbuf[b, pl.ds(j * 16, 16)]
                    dv = dbuf[b, pl.ds(j * 16, 16)]
                    plsc.addupdate_scatter(tbl0, [sv], one16)
                    plsc.addupdate_scatter(tbl1, [dv], one16)
        return carry

    lax.fori_loop(0, (cnt + 4) // 4, superstep, 0)
    pltpu.sync_copy(tbl0, out_hbm.at[pl.ds(wid * NP, NP)])
    pltpu.sync_copy(tbl1, out_hbm.at[pl.ds((NW + wid) * NP, NP)])


def _sc_degrees(src, dst):
    k = pl.kernel(
        _deg_body,
        out_type=jax.ShapeDtypeStruct((2 * NW * NP,), jnp.float32),
        mesh=plsc.VectorSubcoreMesh(core_axis_name="c", subcore_axis_name="s"),
        compiler_params=pltpu.CompilerParams(needs_layout_passes=False),
        scratch_types=[
            pltpu.VMEM((4, K), jnp.int32),
            pltpu.VMEM((4, K), jnp.int32),
            pltpu.VMEM((NP,), jnp.float32),
            pltpu.VMEM((NP,), jnp.float32),
            pltpu.SemaphoreType.DMA,
            pltpu.SemaphoreType.DMA,
            pltpu.SemaphoreType.DMA,
            pltpu.SemaphoreType.DMA,
        ],
    )
    return k(src, dst).reshape(2, NW, NP)


def _agg_body(h_hbm, src_hbm, dst_hbm, zeros_hbm, out_hbm,
              sidx, didx, rows,
              gs0, gs1, ss0, ss1, is0, is1, is2, is3, agg_sh):
    gsem = (gs0, gs1)
    ssem = (ss0, ss1)
    isem = (is0, is1, is2, is3)
    cid, sid, start, cnt = _tile_ids()
    r0 = sid * RPT

    def idx_start(t, i):
        pltpu.async_copy(src_hbm.at[pl.ds((start + t) * K, K)], sidx.at[i], isem[i])
        pltpu.async_copy(dst_hbm.at[pl.ds((start + t) * K, K)], didx.at[i], isem[i])

    def idx_wait(t, i):
        pltpu.make_async_copy(src_hbm.at[pl.ds((start + t) * K, K)], sidx.at[i],
                              isem[i]).wait()
        pltpu.make_async_copy(dst_hbm.at[pl.ds((start + t) * K, K)], didx.at[i],
                              isem[i]).wait()

    pltpu.sync_copy(zeros_hbm.at[pl.ds(r0, RPT)], agg_sh.at[pl.ds(r0, RPT)])
    idx_start(0, 0)
    idx_start(1, 1)
    plsc.subcore_barrier()

    def superstep(s, carry):
        for b in range(4):
            t = s * 4 + b
            r = b % 2
            pr = (b - 1) % 2
            pi = (b - 1) % 4
            ni = (b + 2) % 4

            @pl.when(t < cnt)
            def _launch():
                @pl.when(t >= 2)
                def _free():
                    pltpu.make_async_copy(rows.at[r], agg_sh.at[didx.at[0]],
                                          ssem[r]).wait()
                idx_wait(t, b)
                pltpu.async_copy(h_hbm.at[sidx.at[b]], rows.at[r], gsem[r])

                @pl.when(t + 2 < cnt)
                def _prefetch():
                    idx_start(t + 2, ni)

            @pl.when(jnp.logical_and(t >= 1, t <= cnt))
            def _consume():
                pltpu.make_async_copy(h_hbm.at[sidx.at[pi]], rows.at[pr],
                                      gsem[pr]).wait()
                pltpu.async_copy(rows.at[pr], agg_sh.at[didx.at[pi]],
                                 ssem[pr], add=True)
        return carry

    lax.fori_loop(0, (cnt + 4) // 4, superstep, 0)
    for r in range(2):
        pltpu.make_async_copy(rows.at[r], agg_sh.at[didx.at[0]], ssem[r]).wait()
    plsc.subcore_barrier()
    pltpu.sync_copy(agg_sh.at[pl.ds(r0, RPT)], out_hbm.at[cid].at[pl.ds(r0, RPT)])


def _sc_aggregate(h, src, dst):
    zeros = jnp.zeros((NP, 128), jnp.float32)
    k = pl.kernel(
        _agg_body,
        out_type=jax.ShapeDtypeStruct((NC, NP, 128), jnp.float32),
        mesh=plsc.VectorSubcoreMesh(core_axis_name="c", subcore_axis_name="s"),
        scratch_types=[
            pltpu.VMEM((4, K), jnp.int32),
            pltpu.VMEM((4, K), jnp.int32),
            pltpu.VMEM((2, K, 128), jnp.float32),
            pltpu.SemaphoreType.DMA,
            pltpu.SemaphoreType.DMA,
            pltpu.SemaphoreType.DMA,
            pltpu.SemaphoreType.DMA,
            pltpu.SemaphoreType.DMA,
            pltpu.SemaphoreType.DMA,
            pltpu.SemaphoreType.DMA,
            pltpu.SemaphoreType.DMA,
            pltpu.VMEM_SHARED((NP, 128), jnp.float32),
        ],
    )
    return k(h, src, dst, zeros)


def _norm_col(deg_t):
    d = jnp.sum(deg_t, axis=0, keepdims=True)
    d = jnp.transpose(d)[:NN, :]
    return lax.rsqrt(jnp.maximum(d, 1.0))


def _nsrc(deg):
    return _norm_col(deg[0])


def _ndst(deg):
    return _norm_col(deg[1])


def _t1_body(deg_ref, x_ref, w1_ref, out_ref):
    deg = deg_ref[...]
    out_ref[...] = jnp.dot(x_ref[...] * _nsrc(deg), w1_ref[...],
                           preferred_element_type=jnp.float32)


def _t2_body(deg_ref, p_ref, b1_ref, out_ref):
    deg = deg_ref[...]
    agg = p_ref[0, :NN, :] + p_ref[1, :NN, :]
    h1 = jnp.maximum(agg * _ndst(deg) + b1_ref[...][None, :], 0.0)
    out_ref[...] = h1 * _nsrc(deg)


def _t3_body(deg_ref, p_ref, b2_ref, w2_ref, out_ref):
    deg = deg_ref[...]
    agg = p_ref[0, :NN, :] + p_ref[1, :NN, :]
    pre = jnp.dot(agg, w2_ref[...], preferred_element_type=jnp.float32)
    out_ref[...] = jnp.maximum(pre * _ndst(deg) + b2_ref[...][None, :], 0.0)


def _tc_call(body, out_shape, *args):
    return pl.pallas_call(body, out_shape=out_shape)(*args)


BM = 200


def _adj_body(a_ref, b_ref, out_ref):
    out_ref[...] = lax.dot_general(
        a_ref[...], b_ref[...], (((1,), (1,)), ((), ())),
        preferred_element_type=jnp.float32)


def _adj(h2):
    return pl.pallas_call(
        _adj_body,
        grid=(NN // BM,),
        in_specs=[pl.BlockSpec((BM, 64), lambda i: (i, 0)),
                  pl.BlockSpec((NN, 64), lambda i: (0, 0))],
        out_specs=pl.BlockSpec((BM, NN), lambda i: (i, 0)),
        out_shape=jax.ShapeDtypeStruct((NN, NN), jnp.float32),
    )(h2, h2)


def kernel(inputs, edge_index, W1, b1, W2, b2):
    src = edge_index[0]
    dst = edge_index[1]
    deg = _sc_degrees(src, dst)
    h1pre = _tc_call(_t1_body, jax.ShapeDtypeStruct((NN, 128), jnp.float32),
                     deg, inputs, W1)
    p1 = _sc_aggregate(h1pre, src, dst)
    h1n = _tc_call(_t2_body, jax.ShapeDtypeStruct((NN, 128), jnp.float32),
                   deg, p1, b1)
    p2 = _sc_aggregate(h1n, src, dst)
    h2 = _tc_call(_t3_body, jax.ShapeDtypeStruct((NN, 64), jnp.float32),
                  deg, p2, b2, W2)
    adj = _adj(h2)
    return (adj, h2)
